# Initial kernel scaffold; baseline (speedup 1.0000x reference)
import jax
import jax.numpy as jnp
from jax.experimental import pallas as pl


def kernel(x, edge_index, edge_attributes, Wl1, Wr1, We1, Wl2, Wr2, We2, att1, b1, att2, b2, Wf, bf):
    raise NotImplementedError("write your pallas kernel here")


# speedup vs baseline: 5.7343x; 5.7343x over previous
import functools

import jax
import jax.numpy as jnp
from jax import lax
from jax.experimental import pallas as pl
from jax.experimental.pallas import tpu as pltpu
from jax.experimental.pallas import tpu_sc as plsc

N = 10000
E = 320000
D = 128

NC = 2
NS = 16
L = 16
NW = NC * NS

CHUNK = 40
NCHUNK = E // CHUNK
ITERS = NCHUNK // NW
DBLK = D // L


def _node_mm_body(x_ref, wl_ref, wr_ref, xl_ref, xr_ref):
    x = x_ref[...]
    xl_ref[...] = jnp.dot(x, wl_ref[...], preferred_element_type=jnp.float32)
    xr_ref[...] = jnp.dot(x, wr_ref[...], preferred_element_type=jnp.float32)


def _node_mm(x, wl, wr):
    return pl.pallas_call(
        _node_mm_body,
        out_shape=(
            jax.ShapeDtypeStruct((N, D), jnp.float32),
            jax.ShapeDtypeStruct((N, D), jnp.float32),
        ),
    )(x, wl, wr)


EB = 3200


def _edge_mm_body(a_ref, w1_ref, w2_ref, o1_ref, o2_ref):
    a = a_ref[...]
    o1_ref[...] = jnp.dot(a, w1_ref[...], preferred_element_type=jnp.float32)
    o2_ref[...] = jnp.dot(a, w2_ref[...], preferred_element_type=jnp.float32)


def _edge_mm(edge_attr, we1, we2):
    grid = (E // EB,)
    blk = pl.BlockSpec((EB, D), lambda i: (i, 0))
    wblk = pl.BlockSpec((D, D), lambda i: (0, 0))
    return pl.pallas_call(
        _edge_mm_body,
        grid=grid,
        in_specs=[blk, wblk, wblk],
        out_specs=(blk, blk),
        out_shape=(
            jax.ShapeDtypeStruct((E, D), jnp.float32),
            jax.ShapeDtypeStruct((E, D), jnp.float32),
        ),
    )(edge_attr, we1, we2)


def _combine_h(numer_ref, den_ref, ones_ref, b_ref):
    num = numer_ref[0] + numer_ref[1]
    den = lax.dot_general(den_ref[...], ones_ref[...],
                          (((0,), (0,)), ((), ())),
                          preferred_element_type=jnp.float32)
    return num / (den + 1e-16) + b_ref[...]


def _mid_mm_body(numer_ref, den_ref, ones_ref, b_ref, wl_ref, wr_ref,
                 xl_ref, xr_ref):
    h = _combine_h(numer_ref, den_ref, ones_ref, b_ref)
    xl_ref[...] = jnp.dot(h, wl_ref[...], preferred_element_type=jnp.float32)
    xr_ref[...] = jnp.dot(h, wr_ref[...], preferred_element_type=jnp.float32)


def _mid_mm(numer, den, b, wl, wr):
    ones = jnp.ones((NW, 1), jnp.float32)
    return pl.pallas_call(
        _mid_mm_body,
        out_shape=(
            jax.ShapeDtypeStruct((N, D), jnp.float32),
            jax.ShapeDtypeStruct((N, D), jnp.float32),
        ),
    )(numer, den, ones, b.reshape(1, D), wl, wr)


K = 10


def _final_body(numer_ref, den_ref, ones_ref, b_ref, wf_ref, bf_ref,
                vals_ref, idxs_ref):
    h = _combine_h(numer_ref, den_ref, ones_ref, b_ref)
    s = jnp.dot(h, wf_ref[...], preferred_element_type=jnp.float32)
    s = s + bf_ref[0, 0]
    iota = lax.broadcasted_iota(jnp.int32, (N, 1), 0)
    lane = lax.broadcasted_iota(jnp.int32, (1, 128), 1)
    vals = jnp.zeros((1, 128), jnp.float32)
    idxs = jnp.zeros((1, 128), jnp.int32)
    for k in range(K):
        m = jnp.max(s)
        idx = jnp.min(jnp.where(s == m, iota, N))
        vals = jnp.where(lane == k, m, vals)
        idxs = jnp.where(lane == k, idx, idxs)
        s = jnp.where(iota == idx, -jnp.inf, s)
    vals_ref[...] = vals
    idxs_ref[...] = idxs


def _final(numer, den, b, wf, bf):
    ones = jnp.ones((NW, 1), jnp.float32)
    vals, idxs = pl.pallas_call(
        _final_body,
        out_shape=(
            jax.ShapeDtypeStruct((1, 128), jnp.float32),
            jax.ShapeDtypeStruct((1, 128), jnp.int32),
        ),
    )(numer, den, ones, b.reshape(1, D), wf, bf.reshape(1, 1))
    return vals[0, :K], idxs[0, :K]


def _rne_bf16(v):
    bits = plsc.bitcast(v, jnp.int32)
    lsb = lax.shift_right_logical(bits, 16) & 1
    r = (bits + jnp.int32(0x7FFF) + lsb) & jnp.int32(-65536)
    return plsc.bitcast(r, jnp.float32)

def _sc_layer_body(xl_hbm, xr_hbm, ea_hbm, src_hbm, dst_hbm, att_hbm,
                   numer_out, denom_out,
                   srcv, dstv, xlv, xrv, eav, exb, attv, denv,
                   numer_sp, sem1, sem2, sem3):
    c = lax.axis_index("c")
    s = lax.axis_index("s")
    w = s * NC + c

    zero16 = jnp.zeros((L,), jnp.float32)

    def _zd(i, carry):
        denv[pl.ds(i * L, L)] = zero16
        return carry
    lax.fori_loop(0, N // L, _zd, 0)

    def _zo(i, carry):
        for j in range(DBLK):
            eav[i, pl.ds(j * L, L)] = zero16
        return carry
    lax.fori_loop(0, CHUNK, _zo, 0)

    stripe0 = 624
    stripe = 640
    for k in range(stripe // CHUNK):
        pltpu.sync_copy(eav.at[pl.ds(0, CHUNK)],
                        numer_sp.at[pl.ds(s * stripe0 + k * CHUNK, CHUNK)])

    plsc.subcore_barrier()

    pltpu.sync_copy(att_hbm, attv)
    att_regs = [_rne_bf16(attv[pl.ds(j * L, L)]) for j in range(DBLK)]
    lane = lax.iota(jnp.int32, L)
    lane0 = lane == 0
    lane_hi = lane >= jnp.int32(3 * L - CHUNK)

    def chunk_body(it, carry):
        cid = w + it * NW
        base = cid * CHUNK
        pltpu.sync_copy(src_hbm.at[pl.ds(base, CHUNK)], srcv)
        pltpu.sync_copy(dst_hbm.at[pl.ds(base, CHUNK)], dstv)
        cp1 = pltpu.async_copy(xl_hbm.at[srcv], xlv, sem1)
        cp2 = pltpu.async_copy(xr_hbm.at[dstv], xrv, sem2)
        cp3 = pltpu.async_copy(ea_hbm.at[pl.ds(base, CHUNK)], eav, sem3)
        cp1.wait()
        cp2.wait()
        cp3.wait()

        def edge_body(e, ecarry):
            acc = jnp.zeros((L,), jnp.float32)
            xl_regs = []
            for j in range(DBLK):
                vxl = xlv[e, pl.ds(j * L, L)]
                z = vxl + xrv[e, pl.ds(j * L, L)] + eav[e, pl.ds(j * L, L)]
                z = jnp.maximum(z, z * 0.2)
                acc = acc + _rne_bf16(z) * att_regs[j]
                xl_regs.append(vxl)
            logit = jnp.sum(acc)
            ex = jnp.exp(jnp.broadcast_to(logit, (L,)))
            for j in range(DBLK):
                eav[e, pl.ds(j * L, L)] = xl_regs[j] * ex
            plsc.store_scatter(exb, [jnp.full((L,), e, jnp.int32)], ex,
                               mask=lane0)
            return ecarry

        lax.fori_loop(0, CHUNK, edge_body, 0)

        pltpu.sync_copy(eav, numer_sp.at[dstv], add=True)

        for g, (off, m) in enumerate([(0, None), (L, None),
                                      (CHUNK - L, lane_hi)]):
            vdst = dstv[pl.ds(off, L)]
            vex = exb[pl.ds(off, L)]
            plsc.addupdate_scatter(denv, [vdst], vex, mask=m)
        return carry

    lax.fori_loop(0, ITERS, chunk_body, 0)

    plsc.subcore_barrier()

    pltpu.sync_copy(numer_sp.at[pl.ds(s * stripe0, stripe)],
                    numer_out.at[c, pl.ds(s * stripe0, stripe)])
    pltpu.sync_copy(denv, denom_out.at[w, 0])


def _sc_layer(xl, xr, ea, src, dst, att):
    mesh = plsc.VectorSubcoreMesh(core_axis_name="c", subcore_axis_name="s",
                                  num_cores=NC, num_subcores=NS)
    kfn = pl.kernel(
        _sc_layer_body,
        out_type=(
            jax.ShapeDtypeStruct((NC, N, D), jnp.float32),
            jax.ShapeDtypeStruct((NW, 1, N), jnp.float32),
        ),
        mesh=mesh,
        scratch_types=[
            pltpu.VMEM((CHUNK,), jnp.int32),
            pltpu.VMEM((CHUNK,), jnp.int32),
            pltpu.VMEM((CHUNK, D), jnp.float32),
            pltpu.VMEM((CHUNK, D), jnp.float32),
            pltpu.VMEM((CHUNK, D), jnp.float32),
            pltpu.VMEM((CHUNK,), jnp.float32),
            pltpu.VMEM((D,), jnp.float32),
            pltpu.VMEM((N,), jnp.float32),
            pltpu.VMEM_SHARED((N, D), jnp.float32),
            pltpu.SemaphoreType.DMA,
            pltpu.SemaphoreType.DMA,
            pltpu.SemaphoreType.DMA,
        ],
        compiler_params=pltpu.CompilerParams(needs_layout_passes=False),
    )
    return kfn(xl, xr, ea, src, dst, att)


def kernel(x, edge_index, edge_attributes, Wl1, Wr1, We1, Wl2, Wr2, We2,
           att1, b1, att2, b2, Wf, bf):
    ei = edge_index.astype(jnp.int32)
    src = ei[0]
    dst = ei[1]

    xl1, xr1 = _node_mm(x, Wl1, Wr1)
    ea1, ea2 = _edge_mm(edge_attributes, We1, We2)

    numer1, den1 = _sc_layer(xl1, xr1, ea1, src, dst, att1)
    xl2, xr2 = _mid_mm(numer1, den1.reshape(NW, N), b1, Wl2, Wr2)

    numer2, den2 = _sc_layer(xl2, xr2, ea2, src, dst, att2)
    return _final(numer2, den2.reshape(NW, N), b2, Wf, bf)


# speedup vs baseline: 9.6483x; 1.6826x over previous
import functools

import jax
import jax.numpy as jnp
from jax import lax
from jax.experimental import pallas as pl
from jax.experimental.pallas import tpu as pltpu
from jax.experimental.pallas import tpu_sc as plsc

N = 10000
E = 320000
D = 128

NC = 2
NS = 16
L = 16
NW = NC * NS

CHUNK = 40
NCHUNK = E // CHUNK
ITERS = NCHUNK // NW
DBLK = D // L


def _node_mm_body(x_ref, wl_ref, wr_ref, xl_ref, xr_ref):
    x = x_ref[...]
    xl_ref[...] = jnp.dot(x, wl_ref[...], preferred_element_type=jnp.float32)
    xr_ref[...] = jnp.dot(x, wr_ref[...], preferred_element_type=jnp.float32)


def _node_mm(x, wl, wr):
    return pl.pallas_call(
        _node_mm_body,
        out_shape=(
            jax.ShapeDtypeStruct((N, D), jnp.float32),
            jax.ShapeDtypeStruct((N, D), jnp.float32),
        ),
    )(x, wl, wr)


EB = 3200


def _edge_mm_body(a_ref, w1_ref, w2_ref, o1_ref, o2_ref):
    a = a_ref[...]
    o1_ref[...] = jnp.dot(a, w1_ref[...], preferred_element_type=jnp.float32)
    o2_ref[...] = jnp.dot(a, w2_ref[...], preferred_element_type=jnp.float32)


def _edge_mm(edge_attr, we1, we2):
    grid = (E // EB,)
    blk = pl.BlockSpec((EB, D), lambda i: (i, 0))
    wblk = pl.BlockSpec((D, D), lambda i: (0, 0))
    return pl.pallas_call(
        _edge_mm_body,
        grid=grid,
        in_specs=[blk, wblk, wblk],
        out_specs=(blk, blk),
        out_shape=(
            jax.ShapeDtypeStruct((E, D), jnp.float32),
            jax.ShapeDtypeStruct((E, D), jnp.float32),
        ),
    )(edge_attr, we1, we2)


def _combine_h(numer_ref, den_ref, ones_ref, b_ref):
    num = numer_ref[0] + numer_ref[1]
    den = lax.dot_general(den_ref[...], ones_ref[...],
                          (((0,), (0,)), ((), ())),
                          preferred_element_type=jnp.float32)
    return num / (den + 1e-16) + b_ref[...]


def _mid_mm_body(numer_ref, den_ref, ones_ref, b_ref, wl_ref, wr_ref,
                 xl_ref, xr_ref):
    h = _combine_h(numer_ref, den_ref, ones_ref, b_ref)
    xl_ref[...] = jnp.dot(h, wl_ref[...], preferred_element_type=jnp.float32)
    xr_ref[...] = jnp.dot(h, wr_ref[...], preferred_element_type=jnp.float32)


def _mid_mm(numer, den, b, wl, wr):
    ones = jnp.ones((NW, 1), jnp.float32)
    return pl.pallas_call(
        _mid_mm_body,
        out_shape=(
            jax.ShapeDtypeStruct((N, D), jnp.float32),
            jax.ShapeDtypeStruct((N, D), jnp.float32),
        ),
    )(numer, den, ones, b.reshape(1, D), wl, wr)


K = 10


def _final_body(numer_ref, den_ref, ones_ref, b_ref, wf_ref, bf_ref,
                vals_ref, idxs_ref):
    h = _combine_h(numer_ref, den_ref, ones_ref, b_ref)
    s = jnp.dot(h, wf_ref[...], preferred_element_type=jnp.float32)
    s = s + bf_ref[0, 0]
    iota = lax.broadcasted_iota(jnp.int32, (N, 1), 0)
    lane = lax.broadcasted_iota(jnp.int32, (1, 128), 1)
    vals = jnp.zeros((1, 128), jnp.float32)
    idxs = jnp.zeros((1, 128), jnp.int32)
    for k in range(K):
        m = jnp.max(s)
        idx = jnp.min(jnp.where(s == m, iota, N))
        vals = jnp.where(lane == k, m, vals)
        idxs = jnp.where(lane == k, idx, idxs)
        s = jnp.where(iota == idx, -jnp.inf, s)
    vals_ref[...] = vals
    idxs_ref[...] = idxs


def _final(numer, den, b, wf, bf):
    ones = jnp.ones((NW, 1), jnp.float32)
    vals, idxs = pl.pallas_call(
        _final_body,
        out_shape=(
            jax.ShapeDtypeStruct((1, 128), jnp.float32),
            jax.ShapeDtypeStruct((1, 128), jnp.int32),
        ),
    )(numer, den, ones, b.reshape(1, D), wf, bf.reshape(1, 1))
    return vals[0, :K], idxs[0, :K]


def _rne_bf16(v):
    bits = plsc.bitcast(v, jnp.int32)
    lsb = lax.shift_right_logical(bits, 16) & 1
    r = (bits + jnp.int32(0x7FFF) + lsb) & jnp.int32(-65536)
    return plsc.bitcast(r, jnp.float32)

def _sc_layer_body(xl_hbm, xr_hbm, ea_hbm, src_hbm, dst_hbm, att_hbm,
                   numer_out, denom_out,
                   srcv0, srcv1, dstv0, dstv1, xlv0, xlv1, xrv0, xrv1,
                   eav0, eav1, exb, attv, denv, numer_sp, *sems):
    c = lax.axis_index("c")
    s = lax.axis_index("s")
    w = s * NC + c

    srcv = [srcv0, srcv1]
    dstv = [dstv0, dstv1]
    xlv = [xlv0, xlv1]
    xrv = [xrv0, xrv1]
    eav = [eav0, eav1]
    sis, sid, sxl, sxr, sea = (sems[0:2], sems[2:4], sems[4:6], sems[6:8],
                               sems[8:10])

    zero16 = jnp.zeros((L,), jnp.float32)

    def _zd(i, carry):
        denv[pl.ds(i * L, L)] = zero16
        return carry
    lax.fori_loop(0, N // L, _zd, 0)

    def _zo(i, carry):
        for j in range(DBLK):
            eav0[i, pl.ds(j * L, L)] = zero16
        return carry
    lax.fori_loop(0, CHUNK, _zo, 0)

    stripe0 = 624
    stripe = 640
    for k in range(stripe // CHUNK):
        pltpu.sync_copy(eav0.at[pl.ds(0, CHUNK)],
                        numer_sp.at[pl.ds(s * stripe0 + k * CHUNK, CHUNK)])

    plsc.subcore_barrier()

    pltpu.sync_copy(att_hbm, attv)
    att_regs = [_rne_bf16(attv[pl.ds(j * L, L)]) for j in range(DBLK)]
    lane = lax.iota(jnp.int32, L)
    lane0 = lane == 0
    lane_hi = lane >= jnp.int32(3 * L - CHUNK)


    def _base(k):
        return (w + jnp.minimum(k, ITERS - 1) * NW) * CHUNK

    def issue_idx(k, b):
        base = _base(k)
        pltpu.async_copy(src_hbm.at[pl.ds(base, CHUNK)], srcv[b], sis[b])
        pltpu.async_copy(dst_hbm.at[pl.ds(base, CHUNK)], dstv[b], sid[b])

    def wait_idx(b):
        pltpu.make_async_copy(src_hbm.at[pl.ds(0, CHUNK)], srcv[b],
                              sis[b]).wait()
        pltpu.make_async_copy(dst_hbm.at[pl.ds(0, CHUNK)], dstv[b],
                              sid[b]).wait()

    def issue_gathers(k, b):
        pltpu.async_copy(xl_hbm.at[srcv[b]], xlv[b], sxl[b])
        pltpu.async_copy(xr_hbm.at[dstv[b]], xrv[b], sxr[b])
        pltpu.async_copy(ea_hbm.at[pl.ds(_base(k), CHUNK)], eav[b], sea[b])

    def wait_gathers(b):
        pltpu.make_async_copy(xl_hbm.at[srcv[b]], xlv[b], sxl[b]).wait()
        pltpu.make_async_copy(xr_hbm.at[dstv[b]], xrv[b], sxr[b]).wait()
        pltpu.make_async_copy(ea_hbm.at[pl.ds(0, CHUNK)], eav[b],
                              sea[b]).wait()

    issue_idx(0, 0)
    wait_idx(0)
    issue_gathers(0, 0)
    issue_idx(1, 1)

    def outer(i2, carry):
        for b in (0, 1):
            k = i2 * 2 + b
            other = 1 - b
            wait_idx(other)
            issue_gathers(k + 1, other)
            wait_gathers(b)

            xlv_b, xrv_b, eav_b = xlv[b], xrv[b], eav[b]

            @plsc.parallel_loop(0, CHUNK, 1, unroll=4)
            def edge_body(e):
                acc = jnp.zeros((L,), jnp.float32)
                xl_regs = []
                for j in range(DBLK):
                    vxl = xlv_b[e, pl.ds(j * L, L)]
                    z = vxl + xrv_b[e, pl.ds(j * L, L)] + eav_b[e, pl.ds(j * L, L)]
                    z = jnp.maximum(z, z * 0.2)
                    acc = acc + _rne_bf16(z) * att_regs[j]
                    xl_regs.append(vxl)
                logit = jnp.sum(acc)
                ex = jnp.exp(jnp.broadcast_to(logit, (L,)))
                for j in range(DBLK):
                    eav_b[e, pl.ds(j * L, L)] = xl_regs[j] * ex
                plsc.store_scatter(exb, [jnp.full((L,), e, jnp.int32)], ex,
                                   mask=lane0)

            pltpu.sync_copy(eav_b, numer_sp.at[dstv[b]], add=True)

            for off, m in [(0, None), (L, None), (CHUNK - L, lane_hi)]:
                vdst = dstv[b][pl.ds(off, L)]
                vex = exb[pl.ds(off, L)]
                plsc.addupdate_scatter(denv, [vdst], vex, mask=m)

            issue_idx(k + 2, b)
        return carry

    lax.fori_loop(0, ITERS // 2, outer, 0)

    wait_gathers(0)
    wait_idx(1)

    plsc.subcore_barrier()

    pltpu.sync_copy(numer_sp.at[pl.ds(s * stripe0, stripe)],
                    numer_out.at[c, pl.ds(s * stripe0, stripe)])
    pltpu.sync_copy(denv, denom_out.at[w, 0])


def _sc_layer(xl, xr, ea, src, dst, att):
    mesh = plsc.VectorSubcoreMesh(core_axis_name="c", subcore_axis_name="s",
                                  num_cores=NC, num_subcores=NS)
    kfn = pl.kernel(
        _sc_layer_body,
        out_type=(
            jax.ShapeDtypeStruct((NC, N, D), jnp.float32),
            jax.ShapeDtypeStruct((NW, 1, N), jnp.float32),
        ),
        mesh=mesh,
        scratch_types=(
            [pltpu.VMEM((CHUNK,), jnp.int32)] * 4
            + [pltpu.VMEM((CHUNK, D), jnp.float32)] * 6
            + [
                pltpu.VMEM((CHUNK,), jnp.float32),
                pltpu.VMEM((D,), jnp.float32),
                pltpu.VMEM((N,), jnp.float32),
                pltpu.VMEM_SHARED((N, D), jnp.float32),
            ]
            + [pltpu.SemaphoreType.DMA] * 10
        ),
        compiler_params=pltpu.CompilerParams(needs_layout_passes=False),
    )
    return kfn(xl, xr, ea, src, dst, att)


def kernel(x, edge_index, edge_attributes, Wl1, Wr1, We1, Wl2, Wr2, We2,
           att1, b1, att2, b2, Wf, bf):
    ei = edge_index.astype(jnp.int32)
    src = ei[0]
    dst = ei[1]

    xl1, xr1 = _node_mm(x, Wl1, Wr1)
    ea1, ea2 = _edge_mm(edge_attributes, We1, We2)

    numer1, den1 = _sc_layer(xl1, xr1, ea1, src, dst, att1)
    xl2, xr2 = _mid_mm(numer1, den1.reshape(NW, N), b1, Wl2, Wr2)

    numer2, den2 = _sc_layer(xl2, xr2, ea2, src, dst, att2)
    return _final(numer2, den2.reshape(NW, N), b2, Wf, bf)
